# baseline (speedup 1.0000x reference)
import functools

import jax
import jax.numpy as jnp
from jax import lax
from jax.experimental import pallas as pl
from jax.experimental.pallas import tpu as pltpu
from jax.experimental.pallas import tpu_sc as plsc

_S = 30.0
_BLOCK = 4096


def _sc_margin_gather(B):
    info = plsc.get_sparse_core_info()
    NC, NS, L = info.num_cores, info.num_subcores, info.num_lanes
    NW = NC * NS
    bw = B // NW
    KC = bw // 128
    mesh = plsc.VectorSubcoreMesh(core_axis_name="c", subcore_axis_name="s")

    @functools.partial(
        pl.kernel,
        mesh=mesh,
        out_type=jax.ShapeDtypeStruct((B,), jnp.float32),
        scratch_types=[
            pltpu.VMEM((bw,), jnp.int32),
            pltpu.VMEM((bw,), jnp.float32),
            pltpu.SemaphoreType.DMA,
        ],
    )
    def k(m_hbm, tgt_hbm, mt_out, tgt_v, mt_v, sem):
        wid = lax.axis_index("s") * NC + lax.axis_index("c")
        base = wid * bw
        pltpu.sync_copy(tgt_hbm.at[pl.ds(base, bw)], tgt_v)
        copies = [
            pltpu.async_copy(m_hbm.at[tgt_v.at[pl.ds(kk * 128, 128)]],
                             mt_v.at[pl.ds(kk * 128, 128)], sem)
            for kk in range(KC)
        ]
        for cp in copies:
            cp.wait()
        pltpu.sync_copy(mt_v, mt_out.at[pl.ds(base, bw)])

    return k


def _dense_kernel(logits_ref, tgt_ref, mx_ref, z_ref, xt_ref):
    x = logits_ref[...]
    t = tgt_ref[...]
    col = lax.broadcasted_iota(jnp.int32, x.shape, 1)
    onehot = col == t
    mx = jnp.max(x, axis=1, keepdims=True)
    z = jnp.sum(jnp.exp(x - mx), axis=1, keepdims=True)
    xt = jnp.sum(jnp.where(onehot, x, 0.0), axis=1, keepdims=True)
    mx_ref[...] = mx
    z_ref[...] = z
    xt_ref[...] = xt


def _combine_kernel(mx_ref, z_ref, xt_ref, mt_ref, out_ref):
    mx = mx_ref[...]
    z = z_ref[...]
    xt = xt_ref[...]
    adj = xt - _S * mt_ref[...]
    zadj = z - jnp.exp(xt - mx) + jnp.exp(adj - mx)
    out_ref[...] = jnp.full((1, 1), jnp.sum(mx + jnp.log(zadj) - adj),
                            jnp.float32)


def kernel(logits, m_list, target):
    B, C = logits.shape
    mt = _sc_margin_gather(B)(m_list, target)
    mx, z, xt = pl.pallas_call(
        _dense_kernel,
        grid=(B // _BLOCK,),
        in_specs=[
            pl.BlockSpec((_BLOCK, C), lambda i: (i, 0)),
            pl.BlockSpec((_BLOCK, 1), lambda i: (i, 0)),
        ],
        out_specs=[
            pl.BlockSpec((_BLOCK, 1), lambda i: (i, 0)),
            pl.BlockSpec((_BLOCK, 1), lambda i: (i, 0)),
            pl.BlockSpec((_BLOCK, 1), lambda i: (i, 0)),
        ],
        out_shape=[
            jax.ShapeDtypeStruct((B, 1), jnp.float32),
            jax.ShapeDtypeStruct((B, 1), jnp.float32),
            jax.ShapeDtypeStruct((B, 1), jnp.float32),
        ],
    )(logits, target.reshape(B, 1))
    sq = int(B ** 0.5)
    out = pl.pallas_call(
        _combine_kernel,
        in_specs=[pl.BlockSpec((sq, sq), lambda: (0, 0))] * 4,
        out_specs=pl.BlockSpec((1, 1), lambda: (0, 0)),
        out_shape=jax.ShapeDtypeStruct((1, 1), jnp.float32),
    )(mx.reshape(sq, sq), z.reshape(sq, sq), xt.reshape(sq, sq),
      mt.reshape(sq, sq))
    return (out[0, 0] / B).astype(jnp.float32)
